# baseline (speedup 1.0000x reference)
import functools

import numpy as np
import jax
import jax.numpy as jnp
from jax import lax
from jax.experimental import pallas as pl
from jax.experimental.pallas import tpu as pltpu
from jax.experimental.pallas import tpu_sc as plsc


def _interp_matrix_T(n, m):
    if m == n:
        return np.eye(n, dtype=np.float32)
    eps = np.float32(np.finfo(np.float32).eps)
    x = np.linspace(0.0, 1.0, n + 2, dtype=np.float32)[1:-1]
    xnew = np.linspace(0.0, 1.0, m + 2, dtype=np.float32)[1:-1]
    ind = np.clip(np.searchsorted(x, xnew) - 1, 0, n - 2)
    dx = (x[1:] - x[:-1]).astype(np.float32)
    a = ((xnew - x[ind]) / (eps + dx[ind])).astype(np.float32)
    A = np.zeros((m, n), np.float32)
    A[np.arange(m), ind] += (1.0 - a).astype(np.float32)
    A[np.arange(m), ind + 1] += a
    return A.T


def _bitonic_stages(n):
    stages = []
    k = 2
    while k <= n:
        j = k // 2
        while j >= 1:
            stages.append((k, j))
            j //= 2
        k *= 2
    return stages


def _mm_body(x_ref, w_ref, o_ref):
    o_ref[0] = jax.lax.dot_general(
        x_ref[0], w_ref[...], (((1,), (1,)), ((), ())),
        preferred_element_type=jnp.float32,
        precision=jax.lax.Precision.HIGHEST)


def _make_fold_body(nb):
    def _fold_body(ys_ref, wt_ref, ref_ref, at_ref, o_ref):
        wtT = jax.lax.dot_general(
            at_ref[...], wt_ref[...], (((1,), (0,)), ((), ())),
            preferred_element_type=jnp.float32,
            precision=jax.lax.Precision.HIGHEST)
        c = jnp.sum(wt_ref[...] * ref_ref[...], axis=0, keepdims=True)
        for b in range(nb):
            acc = jnp.sum(ys_ref[b] * wtT, axis=0, keepdims=True)
            o_ref[pl.ds(b, 1), :] = c - acc
    return _fold_body


def _bitonic_sort_cols(x):
    n, lanes = x.shape
    k = 2
    while k <= n:
        j = k // 2
        while j >= 1:
            g = n // (2 * j)
            x4 = x.reshape(g, 2, j, lanes)
            a, b = x4[:, 0], x4[:, 1]
            mn = jnp.minimum(a, b)
            mx = jnp.maximum(a, b)
            gi = jax.lax.broadcasted_iota(jnp.int32, (g, 1, 1), 0)
            desc = ((gi // (k // (2 * j))) % 2) == 1
            lo = jnp.where(desc, mx, mn)
            hi = jnp.where(desc, mn, mx)
            x = jnp.stack([lo, hi], axis=1).reshape(n, lanes)
            j //= 2
        k *= 2
    return x


def _tc_mm_sort_body(x_ref, w_ref, o_ref):
    xs = jax.lax.dot_general(
        x_ref[0], w_ref[...], (((1,), (1,)), ((), ())),
        preferred_element_type=jnp.float32,
        precision=jax.lax.Precision.HIGHEST)
    o_ref[0] = _bitonic_sort_cols(xs)


def _make_sc_sort(B, N, P):
    info = plsc.get_sparse_core_info()
    nc, ns = info.num_cores, info.num_subcores
    nw = nc * ns
    lanes = info.num_lanes
    sb_lanes = 128
    ngrp = sb_lanes // lanes
    total_cols = B * P
    sb_per_w = total_cols // (nw * sb_lanes)
    npairs = N // 2

    stages = []
    k = 2
    while k <= N:
        stages.append(("rev", k))
        j = k // 4
        while j >= 1:
            stages.append(("str", j))
            j //= 2
        k *= 2

    BLK = 32
    nblk = N // BLK

    def _rev_pairs(k, n):
        klog = k.bit_length() - 1
        out = []
        for i in range(n // 2):
            blk = i >> (klog - 1)
            t = i & (k // 2 - 1)
            base = blk << klog
            out.append((base + t, base + k - 1 - t))
        return out

    def _str_pairs(j, n):
        jlog = j.bit_length() - 1
        out = []
        for i in range(n // 2):
            r1 = ((i >> jlog) << (jlog + 1)) | (i & (j - 1))
            out.append((r1, r1 + j))
        return out

    stages32 = []
    k = 2
    while k <= BLK:
        stages32.append(_rev_pairs(k, BLK))
        j = k // 4
        while j >= 1:
            stages32.append(_str_pairs(j, BLK))
            j //= 2
        k *= 2
    tail32 = [_str_pairs(j, BLK) for j in (16, 8, 4, 2, 1)]

    mesh = plsc.VectorSubcoreMesh(core_axis_name="c", subcore_axis_name="s")

    @functools.partial(
        pl.kernel,
        out_type=jax.ShapeDtypeStruct((B, N, P), jnp.float32),
        mesh=mesh,
        scratch_types=[pltpu.VMEM((N, sb_lanes), jnp.float32)],
    )
    def sc_sort(xs_hbm, ys_hbm, buf):
        wid = lax.axis_index("s") * nc + lax.axis_index("c")

        def _block_apply(base_row, sl, stage_list):
            v = [buf[base_row + r, sl] for r in range(BLK)]
            for pairs in stage_list:
                for r1, r2 in pairs:
                    a, b = v[r1], v[r2]
                    v[r1] = jnp.minimum(a, b)
                    v[r2] = jnp.maximum(a, b)
            for r in range(BLK):
                buf[base_row + r, sl] = v[r]

        def per_sb(s, _):
            col0 = (wid * sb_per_w + s) * sb_lanes
            b = col0 // P
            p0 = col0 % P
            pltpu.sync_copy(xs_hbm.at[b, :, pl.ds(p0, sb_lanes)], buf)

            def per_lane_group(l, _):
                sl = pl.ds(l * lanes, lanes)

                def p1(blk_i, _):
                    _block_apply(blk_i * BLK, sl, stages32)
                    return 0

                lax.fori_loop(0, nblk, p1, 0)

                k = 2 * BLK
                while k <= N:
                    klog = k.bit_length() - 1

                    @plsc.parallel_loop(0, npairs, unroll=4)
                    def _rev(i, _klog=klog, _k=k):
                        blk_ = i >> (_klog - 1)
                        t = i & ((_k // 2) - 1)
                        base = blk_ << _klog
                        r1 = base + t
                        r2 = base + (_k - 1) - t
                        a = buf[r1, sl]
                        bb = buf[r2, sl]
                        buf[r1, sl] = jnp.minimum(a, bb)
                        buf[r2, sl] = jnp.maximum(a, bb)

                    j = k // 4
                    while j >= BLK:
                        jlog = j.bit_length() - 1

                        @plsc.parallel_loop(0, npairs, unroll=4)
                        def _str(i, _j=j, _jlog=jlog):
                            r1 = ((i >> _jlog) << (_jlog + 1)) | (i & (_j - 1))
                            r2 = r1 + _j
                            a = buf[r1, sl]
                            bb = buf[r2, sl]
                            buf[r1, sl] = jnp.minimum(a, bb)
                            buf[r2, sl] = jnp.maximum(a, bb)

                        j //= 2

                    def bp(blk_i, _):
                        _block_apply(blk_i * BLK, sl, tail32)
                        return 0

                    lax.fori_loop(0, nblk, bp, 0)
                    k *= 2
                return 0

            lax.fori_loop(0, ngrp, per_lane_group, 0)
            pltpu.sync_copy(buf, ys_hbm.at[b, :, pl.ds(p0, sb_lanes)])
            return 0

        lax.fori_loop(0, sb_per_w, per_sb, 0)

    return sc_sort


def kernel(X, W, ref, weight):
    B, N, D = X.shape
    M, P = ref.shape
    AT = jnp.asarray(_interp_matrix_T(N, M))
    weightT = weight.T

    B1 = B // 4
    B2 = B - B1

    def _mm(xpart, nb):
        return pl.pallas_call(
            _mm_body,
            grid=(nb,),
            in_specs=[
                pl.BlockSpec((1, N, D), lambda b: (b, 0, 0)),
                pl.BlockSpec((P, D), lambda b: (0, 0)),
            ],
            out_specs=pl.BlockSpec((1, N, P), lambda b: (b, 0, 0)),
            out_shape=jax.ShapeDtypeStruct((nb, N, P), jnp.float32),
        )(xpart, W)

    xs1 = _mm(X[:B1], B1)

    ys1 = _make_sc_sort(B1, N, P)(xs1)

    ys2 = pl.pallas_call(
        _tc_mm_sort_body,
        grid=(B2,),
        in_specs=[
            pl.BlockSpec((1, N, D), lambda b: (b, 0, 0)),
            pl.BlockSpec((P, D), lambda b: (0, 0)),
        ],
        out_specs=pl.BlockSpec((1, N, P), lambda b: (b, 0, 0)),
        out_shape=jax.ShapeDtypeStruct((B2, N, P), jnp.float32),
    )(X[B1:], W)

    pb = 256

    def _fold(ys, nb):
        return pl.pallas_call(
            _make_fold_body(nb),
            grid=(P // pb,),
            in_specs=[
                pl.BlockSpec((nb, N, pb), lambda j: (0, 0, j)),
                pl.BlockSpec((M, pb), lambda j: (0, j)),
                pl.BlockSpec((M, pb), lambda j: (0, j)),
                pl.BlockSpec((N, M), lambda j: (0, 0)),
            ],
            out_specs=pl.BlockSpec((nb, pb), lambda j: (0, j)),
            out_shape=jax.ShapeDtypeStruct((nb, P), jnp.float32),
        )(ys, weightT, ref, AT)

    out2 = _fold(ys2, B2)
    out1 = _fold(ys1, B1)
    return jnp.concatenate([out1, out2], axis=0)


# speedup vs baseline: 1.0777x; 1.0777x over previous
import functools

import numpy as np
import jax
import jax.numpy as jnp
from jax import lax
from jax.experimental import pallas as pl
from jax.experimental.pallas import tpu as pltpu
from jax.experimental.pallas import tpu_sc as plsc


def _interp_matrix_T(n, m):
    if m == n:
        return np.eye(n, dtype=np.float32)
    eps = np.float32(np.finfo(np.float32).eps)
    x = np.linspace(0.0, 1.0, n + 2, dtype=np.float32)[1:-1]
    xnew = np.linspace(0.0, 1.0, m + 2, dtype=np.float32)[1:-1]
    ind = np.clip(np.searchsorted(x, xnew) - 1, 0, n - 2)
    dx = (x[1:] - x[:-1]).astype(np.float32)
    a = ((xnew - x[ind]) / (eps + dx[ind])).astype(np.float32)
    A = np.zeros((m, n), np.float32)
    A[np.arange(m), ind] += (1.0 - a).astype(np.float32)
    A[np.arange(m), ind + 1] += a
    return A.T


def _mm_body(x_ref, w_ref, o_ref):
    o_ref[0] = jax.lax.dot_general(
        x_ref[0], w_ref[...], (((1,), (1,)), ((), ())),
        preferred_element_type=jnp.float32,
        precision=jax.lax.Precision.HIGHEST)


def _make_fold_body(nb1, nb2):
    def _fold_body(ys1_ref, ys2_ref, wt_ref, ref_ref, at_ref, o_ref):
        wtT = jax.lax.dot_general(
            at_ref[...], wt_ref[...], (((1,), (0,)), ((), ())),
            preferred_element_type=jnp.float32,
            precision=jax.lax.Precision.HIGHEST)
        c = jnp.sum(wt_ref[...] * ref_ref[...], axis=0, keepdims=True)
        for b in range(nb1):
            acc = jnp.sum(ys1_ref[b] * wtT, axis=0, keepdims=True)
            o_ref[pl.ds(b, 1), :] = c - acc
        for b in range(nb2):
            acc = jnp.sum(ys2_ref[b] * wtT, axis=0, keepdims=True)
            o_ref[pl.ds(nb1 + b, 1), :] = c - acc
    return _fold_body


def _bitonic_sort_cols(x):
    n, lanes = x.shape
    k = 2
    while k <= n:
        j = k // 2
        while j >= 1:
            g = n // (2 * j)
            x4 = x.reshape(g, 2, j, lanes)
            a, b = x4[:, 0], x4[:, 1]
            mn = jnp.minimum(a, b)
            mx = jnp.maximum(a, b)
            gi = jax.lax.broadcasted_iota(jnp.int32, (g, 1, 1), 0)
            desc = ((gi // (k // (2 * j))) % 2) == 1
            lo = jnp.where(desc, mx, mn)
            hi = jnp.where(desc, mn, mx)
            x = jnp.stack([lo, hi], axis=1).reshape(n, lanes)
            j //= 2
        k *= 2
    return x


def _tc_mm_sort_body(x_ref, w_ref, o_ref):
    xs = jax.lax.dot_general(
        x_ref[0], w_ref[...], (((1,), (1,)), ((), ())),
        preferred_element_type=jnp.float32,
        precision=jax.lax.Precision.HIGHEST)
    o_ref[0] = _bitonic_sort_cols(xs)


def _make_sc_sort(B, N, P):
    info = plsc.get_sparse_core_info()
    nc, ns = info.num_cores, info.num_subcores
    nw = nc * ns
    lanes = info.num_lanes
    sb_lanes = 128
    ngrp = sb_lanes // lanes
    total_cols = B * P
    sb_per_w = total_cols // (nw * sb_lanes)
    npairs = N // 2

    BLK = 32
    nblk = N // BLK

    def _rev_pairs(k, n):
        klog = k.bit_length() - 1
        out = []
        for i in range(n // 2):
            blk = i >> (klog - 1)
            t = i & (k // 2 - 1)
            base = blk << klog
            out.append((base + t, base + k - 1 - t))
        return out

    def _str_pairs(j, n):
        jlog = j.bit_length() - 1
        out = []
        for i in range(n // 2):
            r1 = ((i >> jlog) << (jlog + 1)) | (i & (j - 1))
            out.append((r1, r1 + j))
        return out

    stages32 = []
    k = 2
    while k <= BLK:
        stages32.append(_rev_pairs(k, BLK))
        j = k // 4
        while j >= 1:
            stages32.append(_str_pairs(j, BLK))
            j //= 2
        k *= 2
    tail32 = [_str_pairs(j, BLK) for j in (16, 8, 4, 2, 1)]

    mesh = plsc.VectorSubcoreMesh(core_axis_name="c", subcore_axis_name="s")

    @functools.partial(
        pl.kernel,
        out_type=jax.ShapeDtypeStruct((B, N, P), jnp.float32),
        mesh=mesh,
        scratch_types=[pltpu.VMEM((N, sb_lanes), jnp.float32)],
    )
    def sc_sort(xs_hbm, ys_hbm, buf):
        wid = lax.axis_index("s") * nc + lax.axis_index("c")

        def _block_apply(base_row, sl, stage_list):
            v = [buf[base_row + r, sl] for r in range(BLK)]
            for pairs in stage_list:
                for r1, r2 in pairs:
                    a, b = v[r1], v[r2]
                    v[r1] = jnp.minimum(a, b)
                    v[r2] = jnp.maximum(a, b)
            for r in range(BLK):
                buf[base_row + r, sl] = v[r]

        def per_sb(s, _):
            col0 = (wid * sb_per_w + s) * sb_lanes
            b = col0 // P
            p0 = col0 % P
            pltpu.sync_copy(xs_hbm.at[b, :, pl.ds(p0, sb_lanes)], buf)

            def per_lane_group(l, _):
                sl = pl.ds(l * lanes, lanes)

                def p1(blk_i, _):
                    _block_apply(blk_i * BLK, sl, stages32)
                    return 0

                lax.fori_loop(0, nblk, p1, 0)

                k = 2 * BLK
                while k <= N:
                    klog = k.bit_length() - 1

                    @plsc.parallel_loop(0, npairs, unroll=4)
                    def _rev(i, _klog=klog, _k=k):
                        blk_ = i >> (_klog - 1)
                        t = i & ((_k // 2) - 1)
                        base = blk_ << _klog
                        r1 = base + t
                        r2 = base + (_k - 1) - t
                        a = buf[r1, sl]
                        bb = buf[r2, sl]
                        buf[r1, sl] = jnp.minimum(a, bb)
                        buf[r2, sl] = jnp.maximum(a, bb)

                    j = k // 4
                    while j >= BLK:
                        jlog = j.bit_length() - 1

                        @plsc.parallel_loop(0, npairs, unroll=4)
                        def _str(i, _j=j, _jlog=jlog):
                            r1 = ((i >> _jlog) << (_jlog + 1)) | (i & (_j - 1))
                            r2 = r1 + _j
                            a = buf[r1, sl]
                            bb = buf[r2, sl]
                            buf[r1, sl] = jnp.minimum(a, bb)
                            buf[r2, sl] = jnp.maximum(a, bb)

                        j //= 2

                    def bp(blk_i, _):
                        _block_apply(blk_i * BLK, sl, tail32)
                        return 0

                    lax.fori_loop(0, nblk, bp, 0)
                    k *= 2
                return 0

            lax.fori_loop(0, ngrp, per_lane_group, 0)
            pltpu.sync_copy(buf, ys_hbm.at[b, :, pl.ds(p0, sb_lanes)])
            return 0

        lax.fori_loop(0, sb_per_w, per_sb, 0)

    return sc_sort


def kernel(X, W, ref, weight):
    B, N, D = X.shape
    M, P = ref.shape
    AT = jnp.asarray(_interp_matrix_T(N, M))
    weightT = weight.T

    B1 = B // 4
    B2 = B - B1

    def _mm(xpart, nb):
        return pl.pallas_call(
            _mm_body,
            grid=(nb,),
            in_specs=[
                pl.BlockSpec((1, N, D), lambda b: (b, 0, 0)),
                pl.BlockSpec((P, D), lambda b: (0, 0)),
            ],
            out_specs=pl.BlockSpec((1, N, P), lambda b: (b, 0, 0)),
            out_shape=jax.ShapeDtypeStruct((nb, N, P), jnp.float32),
        )(xpart, W)

    xs1 = _mm(X[:B1], B1)

    ys1 = _make_sc_sort(B1, N, P)(xs1)

    ys2 = pl.pallas_call(
        _tc_mm_sort_body,
        grid=(B2,),
        in_specs=[
            pl.BlockSpec((1, N, D), lambda b: (b, 0, 0)),
            pl.BlockSpec((P, D), lambda b: (0, 0)),
        ],
        out_specs=pl.BlockSpec((1, N, P), lambda b: (b, 0, 0)),
        out_shape=jax.ShapeDtypeStruct((B2, N, P), jnp.float32),
    )(X[B1:], W)

    pb = 256
    out = pl.pallas_call(
        _make_fold_body(B1, B2),
        grid=(P // pb,),
        in_specs=[
            pl.BlockSpec((B1, N, pb), lambda j: (0, 0, j)),
            pl.BlockSpec((B2, N, pb), lambda j: (0, 0, j)),
            pl.BlockSpec((M, pb), lambda j: (0, j)),
            pl.BlockSpec((M, pb), lambda j: (0, j)),
            pl.BlockSpec((N, M), lambda j: (0, 0)),
        ],
        out_specs=pl.BlockSpec((B, pb), lambda j: (0, j)),
        out_shape=jax.ShapeDtypeStruct((B, P), jnp.float32),
    )(ys1, ys2, weightT, ref, AT)
    return out
